# baseline (speedup 1.0000x reference)
import jax
import jax.numpy as jnp
from jax import lax
from jax.experimental import pallas as pl
from jax.experimental.pallas import tpu as pltpu
from jax.experimental.pallas import tpu_sc as plsc

_B = 16384
_D = 64
_NC = 2
_NS = 16
_NW = _NC * _NS
_BPW = _B // _NW
_CHUNK = 128
_NCH = _BPW // _CHUNK
_L = 16


def _sqrt_vec(x):
    i = lax.bitcast_convert_type(x, jnp.int32)
    i = jnp.int32(0x5F3759DF) - lax.shift_right_arithmetic(i, 1)
    y = lax.bitcast_convert_type(i, jnp.float32)
    half_x = x * 0.5
    for _ in range(3):
        y = y * (1.5 - half_x * y * y)
    return jnp.where(x > 0.0, x * y, 0.0)


def _body(heads_hbm, rels_hbm, tails_hbm, ent_hbm, rel_hbm, out_hbm,
          hidx, ridx, tidx, hrow, rrow, trow, outv, sem):
    cid = lax.axis_index("c")
    sid = lax.axis_index("s")
    wid = sid * _NC + cid
    base = wid * _BPW

    for c in range(_NCH):
        off = base + c * _CHUNK
        pltpu.sync_copy(heads_hbm.at[pl.ds(off, _CHUNK)], hidx.at[c])
        pltpu.sync_copy(rels_hbm.at[pl.ds(off, _CHUNK)], ridx.at[c])
        pltpu.sync_copy(tails_hbm.at[pl.ds(off, _CHUNK)], tidx.at[c])

    copies = []
    for c in range(_NCH):
        dst = pl.ds(c * _CHUNK, _CHUNK)
        copies.append(pltpu.async_copy(ent_hbm.at[hidx.at[c]], hrow.at[dst], sem))
        copies.append(pltpu.async_copy(rel_hbm.at[ridx.at[c]], rrow.at[dst], sem))
        copies.append(pltpu.async_copy(ent_hbm.at[tidx.at[c]], trow.at[dst], sem))
    for cp in copies:
        cp.wait()

    lanes = lax.iota(jnp.int32, _L)
    perms = [(lanes + s) & (_L - 1) for s in (8, 4, 2, 1)]

    dnums = lax.GatherDimensionNumbers(
        offset_dims=(), collapsed_slice_dims=(0,), start_index_map=(0,))

    def perm(v, p):
        return lax.gather(v, p[:, None], dnums, (1,),
                          mode=lax.GatherScatterMode.PROMISE_IN_BOUNDS)

    def hsum(v):
        for p in perms:
            v = v + perm(v, p)
        return v

    def group(g, carry):
        totals = jnp.zeros((_L,), jnp.float32)
        for l in range(_L):
            e = g * _L + l
            acc = jnp.zeros((_L,), jnp.float32)
            for k in range(_D // _L):
                h = hrow[e, pl.ds(k * _L, _L)]
                r = rrow[e, pl.ds(k * _L, _L)]
                t = trow[e, pl.ds(k * _L, _L)]
                s = (h + r) - t
                acc = acc + s * s
            totals = jnp.where(lanes == l, hsum(acc), totals)
        outv[pl.ds(g * _L, _L)] = _sqrt_vec(totals)
        return carry

    lax.fori_loop(0, _BPW // _L, group, 0)
    pltpu.sync_copy(outv, out_hbm.at[pl.ds(base, _BPW)])


def kernel(heads, relations, tails, entity_emb, relation_emb):
    mesh = plsc.VectorSubcoreMesh(core_axis_name="c", subcore_axis_name="s")
    f = pl.kernel(
        _body,
        mesh=mesh,
        compiler_params=pltpu.CompilerParams(use_tc_tiling_on_sc=False),
        out_type=jax.ShapeDtypeStruct((_B,), jnp.float32),
        scratch_types=[
            pltpu.VMEM((_NCH, _CHUNK), jnp.int32),
            pltpu.VMEM((_NCH, _CHUNK), jnp.int32),
            pltpu.VMEM((_NCH, _CHUNK), jnp.int32),
            pltpu.VMEM((_BPW, _D), jnp.float32),
            pltpu.VMEM((_BPW, _D), jnp.float32),
            pltpu.VMEM((_BPW, _D), jnp.float32),
            pltpu.VMEM((_BPW,), jnp.float32),
            pltpu.SemaphoreType.DMA,
        ],
    )
    return f(heads, relations, tails, entity_emb, relation_emb)
